# baseline (speedup 1.0000x reference)
import functools

import jax
import jax.numpy as jnp
from jax import lax
from jax.experimental import pallas as pl
from jax.experimental.pallas import tpu as pltpu
from jax.experimental.pallas import tpu_sc as plsc

_B, _C, _H, _W = 8, 1, 512, 512
_N = _B * _C * _H * _W
_NC, _NS = 2, 16
_NW = _NC * _NS
_ROWS_PER_W = _H // 4
_CHUNK_ROWS = 16


def _xent_vec(o, l):
    t = jnp.exp(-jnp.abs(o))
    u = t / (2.0 + t)
    s = u * u
    softplus = u * (2.0 + s * (2.0 / 3.0 + s * (2.0 / 5.0 + s * (2.0 / 7.0 + s * (2.0 / 9.0)))))
    return jnp.maximum(o, 0.0) - jnp.where(l >= 0.5, o, 0.0) + softplus


def _sc_body(o_hbm, l_hbm, out_hbm, o_buf, l_buf, acc_buf):
    wid = lax.axis_index("s") * _NC + lax.axis_index("c")
    b = wid // 4
    r0 = (wid % 4) * _ROWS_PER_W

    acc = jnp.zeros((16,), jnp.float32)
    for c in range(_ROWS_PER_W // _CHUNK_ROWS):
        r = r0 + c * _CHUNK_ROWS
        pltpu.sync_copy(o_hbm.at[b, 0, pl.ds(r, _CHUNK_ROWS), :], o_buf)
        pltpu.sync_copy(l_hbm.at[b, 0, pl.ds(r, _CHUNK_ROWS), :], l_buf)

        def row_body(i, acc):
            def col_body(j, acc):
                o = o_buf[i, pl.ds(j * 16, 16)]
                l = l_buf[i, pl.ds(j * 16, 16)]
                return acc + _xent_vec(o, l)

            return lax.fori_loop(0, _W // 16, col_body, acc)

        acc = lax.fori_loop(0, _CHUNK_ROWS, row_body, acc)

    acc_buf[...] = acc
    pltpu.sync_copy(acc_buf, out_hbm.at[wid])


@functools.partial(jax.jit, static_argnames=())
def _sc_partials(output, label):
    mesh = plsc.VectorSubcoreMesh(core_axis_name="c", subcore_axis_name="s")
    return pl.kernel(
        _sc_body,
        mesh=mesh,
        out_type=jax.ShapeDtypeStruct((_NW, 16), jnp.float32),
        scratch_types=[
            pltpu.VMEM((_CHUNK_ROWS, _W), jnp.float32),
            pltpu.VMEM((_CHUNK_ROWS, _W), jnp.float32),
            pltpu.VMEM((16,), jnp.float32),
        ],
    )(output, label)


def kernel(output, label):
    partials = _sc_partials(output, label)
    return jnp.sum(partials) * jnp.float32(1.0 / _N)
